# Initial kernel scaffold; baseline (speedup 1.0000x reference)
import jax
import jax.numpy as jnp
from jax.experimental import pallas as pl


def kernel(msi_image, hsi_image, e1_w1, e1_b1, e1_a1, e1_w2, e1_b2, e1_a2, e2_w1, e2_b1, e2_a1, e2_w2, e2_b2, e2_a2, fc_w, attn_l, attn_r, gat_b, u_w1, u_b1, u_a1, u_w2, u_b2, u_a2):
    raise NotImplementedError("write your pallas kernel here")


# speedup vs baseline: 3.8610x; 3.8610x over previous
import jax
import jax.numpy as jnp
from jax.experimental import pallas as pl
from jax.experimental.pallas import tpu as pltpu

B, C, H, W = 4, 128, 64, 64
K = 5
HW = H * W
N = B * HW
RB = 512
NRB = HW // RB

_INTERPRET = False


def _prelu(x, a):
    return jnp.where(x >= 0, x, a * x)


def _embed_kernel(hsi_ref, msi_ref,
                  e1w1_ref, e1b1_ref, e1a1_ref, e1w2_ref, e1b2_ref, e1a2_ref,
                  e2w1_ref, e2b1_ref, e2a1_ref, e2w2_ref, e2b2_ref, e2a2_ref,
                  fcl_ref, fcr_ref, al_ref, ar_ref,
                  hf_ref, mf_ref, h_ref, el_ref, er_ref):
    x_h = hsi_ref[0]
    x_m = msi_ref[0]
    hf = _prelu(jnp.dot(x_h, e1w1_ref[:], preferred_element_type=jnp.float32)
                + e1b1_ref[:], e1a1_ref[0, 0])
    hf = _prelu(jnp.dot(hf, e1w2_ref[:], preferred_element_type=jnp.float32)
                + e1b2_ref[:], e1a2_ref[0, 0])
    mf = _prelu(jnp.dot(x_m, e2w1_ref[:], preferred_element_type=jnp.float32)
                + e2b1_ref[:], e2a1_ref[0, 0])
    mf = _prelu(jnp.dot(mf, e2w2_ref[:], preferred_element_type=jnp.float32)
                + e2b2_ref[:], e2a2_ref[0, 0])
    hf_ref[0] = hf
    mf_ref[0] = mf
    h = (jnp.dot(hf, fcl_ref[:], preferred_element_type=jnp.float32)
         + jnp.dot(mf, fcr_ref[:], preferred_element_type=jnp.float32))
    h_ref[0] = h
    el_ref[0] = jnp.sum(h * al_ref[:], axis=-1).reshape(1, HW)
    er_ref[0] = jnp.sum(h * ar_ref[:], axis=-1).reshape(1, HW)


def _topk_kernel(hf_ref, mf_ref, idx_ref):
    hf_b = hf_ref[0]
    mf = mf_ref[0]
    s = jax.lax.dot_general(hf_b, mf, (((1,), (1,)), ((), ())),
                            preferred_element_type=jnp.float32)
    cols = jax.lax.broadcasted_iota(jnp.int32, (RB, HW), 1)
    work = s
    picks = []
    for _ in range(K):
        mx = jnp.max(work, axis=1, keepdims=True)
        cand = jnp.where(work == mx, cols, HW)
        ik = jnp.min(cand, axis=1)
        picks.append(ik.reshape(1, RB))
        work = jnp.where(cols == ik[:, None], -jnp.inf, work)
    idx_ref[0] = jnp.concatenate(picks, axis=0)


def _conv_kernel(x_ref, iden_ref, w1_ref, b1_ref, a1_ref, w2_ref, b2_ref, a2_ref,
                 out_ref, pad_ref):
    x = x_ref[0]
    y1 = jnp.zeros((HW, C), dtype=jnp.float32)
    for t in range(9):
        dy, dx = t // 3, t % 3
        xs = x[dy:dy + H, dx:dx + W, :].reshape(HW, C)
        y1 = y1 + jnp.dot(xs, w1_ref[t], preferred_element_type=jnp.float32)
    y1 = _prelu(y1 + b1_ref[:], a1_ref[0, 0])
    pad_ref[:] = jnp.zeros((H + 2, W + 2, C), dtype=jnp.float32)
    pad_ref[1:H + 1, 1:W + 1, :] = y1.reshape(H, W, C)
    x2 = pad_ref[:]
    y2 = jnp.zeros((HW, C), dtype=jnp.float32)
    for t in range(9):
        dy, dx = t // 3, t % 3
        xs = x2[dy:dy + H, dx:dx + W, :].reshape(HW, C)
        y2 = y2 + jnp.dot(xs, w2_ref[t], preferred_element_type=jnp.float32)
    y2 = _prelu(y2 + b2_ref[:], a2_ref[0, 0])
    out_ref[0] = y2 + iden_ref[0]


def kernel(msi_image, hsi_image, e1_w1, e1_b1, e1_a1, e1_w2, e1_b2, e1_a2,
           e2_w1, e2_b1, e2_a1, e2_w2, e2_b2, e2_a2, fc_w, attn_l, attn_r,
           gat_b, u_w1, u_b1, u_a1, u_w2, u_b2, u_a2):
    hsi_up = jax.image.resize(hsi_image, (B, C, H, W), method='bicubic')
    hsi_t = hsi_up.reshape(B, C, HW).transpose(0, 2, 1)
    msi_t = msi_image.reshape(B, C, HW).transpose(0, 2, 1)

    wspec = pl.BlockSpec((C, C), lambda b: (0, 0))
    bspec = pl.BlockSpec((1, C), lambda b: (0, 0))
    aspec = pl.BlockSpec((1, 1), lambda b: (0, 0))
    xspec = pl.BlockSpec((1, HW, C), lambda b: (b, 0, 0))

    hf, mf, h, el, er = pl.pallas_call(
        _embed_kernel,
        grid=(B,),
        in_specs=[xspec, xspec,
                  wspec, bspec, aspec, wspec, bspec, aspec,
                  wspec, bspec, aspec, wspec, bspec, aspec,
                  wspec, wspec, bspec, bspec],
        out_specs=[pl.BlockSpec((1, HW, C), lambda b: (b, 0, 0)),
                   pl.BlockSpec((1, HW, C), lambda b: (b, 0, 0)),
                   pl.BlockSpec((1, HW, C), lambda b: (b, 0, 0)),
                   pl.BlockSpec((1, 1, HW), lambda b: (b, 0, 0)),
                   pl.BlockSpec((1, 1, HW), lambda b: (b, 0, 0))],
        out_shape=[jax.ShapeDtypeStruct((B, HW, C), jnp.float32),
                   jax.ShapeDtypeStruct((B, HW, C), jnp.float32),
                   jax.ShapeDtypeStruct((B, HW, C), jnp.float32),
                   jax.ShapeDtypeStruct((B, 1, HW), jnp.float32),
                   jax.ShapeDtypeStruct((B, 1, HW), jnp.float32)],
        interpret=_INTERPRET,
    )(hsi_t, msi_t,
      e1_w1.T, e1_b1.reshape(1, C), e1_a1.reshape(1, 1),
      e1_w2.T, e1_b2.reshape(1, C), e1_a2.reshape(1, 1),
      e2_w1.T, e2_b1.reshape(1, C), e2_a1.reshape(1, 1),
      e2_w2.T, e2_b2.reshape(1, C), e2_a2.reshape(1, 1),
      fc_w[:, :C].T, fc_w[:, C:].T,
      attn_l.reshape(1, C), attn_r.reshape(1, C))

    idx = pl.pallas_call(
        _topk_kernel,
        grid=(B, NRB),
        in_specs=[pl.BlockSpec((1, RB, C), lambda b, r: (b, r, 0)),
                  pl.BlockSpec((1, HW, C), lambda b, r: (b, 0, 0))],
        out_specs=pl.BlockSpec((1, K, RB), lambda b, r: (b, 0, r)),
        out_shape=jax.ShapeDtypeStruct((B, K, HW), jnp.int32),
        interpret=_INTERPRET,
    )(hf, mf)

    h_flat = h.reshape(N, C)
    el_flat = el.reshape(N)
    er_flat = er.reshape(N)
    idx_flat = idx.transpose(0, 2, 1).reshape(-1)

    src = jnp.tile(jnp.arange(N, dtype=jnp.int32), K)
    loop = jnp.arange(N, dtype=jnp.int32)
    src = jnp.concatenate([src, loop])
    dst = jnp.concatenate([idx_flat, loop])
    e = el_flat[src] + er_flat[dst]
    e = jnp.where(e > 0, e, 0.2 * e)
    m = jax.ops.segment_max(e, dst, num_segments=N)
    ex = jnp.exp(e - m[dst])
    den = jax.ops.segment_sum(ex, dst, num_segments=N)
    alpha = ex / den[dst]
    rst = jax.ops.segment_sum(alpha[:, None] * h_flat[src], dst,
                              num_segments=N) + gat_b

    x_img = rst.reshape(B, H, W, C)
    x_pad = jnp.pad(x_img, ((0, 0), (1, 1), (1, 1), (0, 0)))
    w1_9 = u_w1.transpose(2, 3, 1, 0).reshape(9, C, C)
    w2_9 = u_w2.transpose(2, 3, 1, 0).reshape(9, C, C)

    y = pl.pallas_call(
        _conv_kernel,
        grid=(B,),
        in_specs=[pl.BlockSpec((1, H + 2, W + 2, C), lambda b: (b, 0, 0, 0)),
                  xspec,
                  pl.BlockSpec((9, C, C), lambda b: (0, 0, 0)),
                  bspec, aspec,
                  pl.BlockSpec((9, C, C), lambda b: (0, 0, 0)),
                  bspec, aspec],
        out_specs=pl.BlockSpec((1, HW, C), lambda b: (b, 0, 0)),
        out_shape=jax.ShapeDtypeStruct((B, HW, C), jnp.float32),
        scratch_shapes=[pltpu.VMEM((H + 2, W + 2, C), jnp.float32)],
        interpret=_INTERPRET,
    )(x_pad, hsi_t, w1_9, u_b1.reshape(1, C), u_a1.reshape(1, 1),
      w2_9, u_b2.reshape(1, C), u_a2.reshape(1, 1))

    return y.reshape(B, H, W, C).transpose(0, 3, 1, 2)
